# Initial kernel scaffold; baseline (speedup 1.0000x reference)
import jax
import jax.numpy as jnp
from jax.experimental import pallas as pl


def kernel(inputs):
    raise NotImplementedError("write your pallas kernel here")


# speedup vs baseline: 127.4594x; 127.4594x over previous
import functools

import jax
import jax.numpy as jnp
from jax import lax
from jax.experimental import pallas as pl
from jax.experimental.pallas import tpu as pltpu
from jax.experimental.pallas import tpu_sc as plsc

_NUM_CORES = 2
_NUM_SUBCORES = 16
_NW = _NUM_CORES * _NUM_SUBCORES
_LANES = 16


@functools.lru_cache(maxsize=None)
def _make_sc_partials(T, blk):
    rows_per_w = T // _NW
    nblk = rows_per_w // blk
    mesh = plsc.VectorSubcoreMesh(core_axis_name="c", subcore_axis_name="s")

    @functools.partial(
        pl.kernel,
        out_type=jax.ShapeDtypeStruct((_NW, 2 * T), jnp.float32),
        mesh=mesh,
        scratch_types=[
            pltpu.VMEM((blk, T), jnp.float32),
            pltpu.VMEM((blk, T), jnp.float32),
            pltpu.VMEM((2 * T,), jnp.float32),
            pltpu.SemaphoreType.DMA,
            pltpu.SemaphoreType.DMA,
        ],
        compiler_params=pltpu.CompilerParams(needs_layout_passes=False),
    )
    def sc_partials(x_hbm, part_hbm, buf0, buf1, acc, sem0, sem1):
        cid = lax.axis_index("c")
        sid = lax.axis_index("s")
        wid = cid * _NUM_SUBCORES + sid
        row0 = wid * rows_per_w

        zero16 = jnp.zeros((_LANES,), jnp.float32)

        def zero_body(k, carry):
            acc[pl.ds(k * _LANES, _LANES)] = zero16
            return carry

        lax.fori_loop(0, (2 * T) // _LANES, zero_body, 0)

        bufs = (buf0, buf1)
        sems = (sem0, sem1)
        iota = lax.iota(jnp.int32, _LANES)

        copies = [None] * nblk
        copies[0] = pltpu.async_copy(x_hbm.at[pl.ds(row0, blk), :], buf0, sem0)
        for b in range(nblk):
            if b + 1 < nblk:
                copies[b + 1] = pltpu.async_copy(
                    x_hbm.at[pl.ds(row0 + (b + 1) * blk, blk), :],
                    bufs[(b + 1) % 2],
                    sems[(b + 1) % 2],
                )
            copies[b].wait()
            buf = bufs[b % 2]

            def row_body(r, carry):
                i = row0 + b * blk + r
                shift = T - i
                lo = jnp.maximum(0, 2 * i - T + 2)

                def chunk_body(k, carry2):
                    jvec = iota + k * _LANES
                    m = (jvec >= lo) & (jvec < T - 1)
                    xv = buf[r, pl.ds(k * _LANES, _LANES)]
                    plsc.addupdate_scatter(acc, [jvec + shift], xv, mask=m)
                    return carry2

                lax.fori_loop(lo // _LANES, T // _LANES, chunk_body, 0)
                return carry

            lax.fori_loop(0, blk, row_body, 0)

        pltpu.sync_copy(acc, part_hbm.at[wid])

    return sc_partials


def _combine_body(T, p_ref, o_ref):
    s = jnp.sum(p_ref[...], axis=0, keepdims=True)
    b = lax.broadcasted_iota(jnp.int32, (1, 2 * T), 1)
    cnt = jnp.where(b < T, 2 * b - T - 1, 2 * T - 1 - b)
    cnt = jnp.maximum(cnt, 0).astype(jnp.float32)
    o_ref[...] = s / cnt


def kernel(inputs):
    T = inputs.shape[0]
    partials = _make_sc_partials(T, 16)(inputs)
    out = pl.pallas_call(
        functools.partial(_combine_body, T),
        out_shape=jax.ShapeDtypeStruct((1, 2 * T), jnp.float32),
    )(partials)
    return out.reshape(2 * T)


# speedup vs baseline: 177.1088x; 1.3895x over previous
import functools

import jax
import jax.numpy as jnp
from jax import lax
from jax.experimental import pallas as pl
from jax.experimental.pallas import tpu as pltpu
from jax.experimental.pallas import tpu_sc as plsc

_NUM_CORES = 2
_NUM_SUBCORES = 16
_NW = _NUM_CORES * _NUM_SUBCORES
_LANES = 16


@functools.lru_cache(maxsize=None)
def _make_sc_partials(T, blk):
    rows_per_w = T // _NW
    nblk = rows_per_w // blk
    mesh = plsc.VectorSubcoreMesh(core_axis_name="c", subcore_axis_name="s")

    @functools.partial(
        pl.kernel,
        out_type=jax.ShapeDtypeStruct((_NW, 2 * T), jnp.float32),
        mesh=mesh,
        scratch_types=[
            pltpu.VMEM((blk, T), jnp.float32),
            pltpu.VMEM((blk, T), jnp.float32),
            pltpu.VMEM((2 * T,), jnp.float32),
            pltpu.SemaphoreType.DMA,
            pltpu.SemaphoreType.DMA,
        ],
        compiler_params=pltpu.CompilerParams(needs_layout_passes=False),
    )
    def sc_partials(x_hbm, part_hbm, buf0, buf1, acc, sem0, sem1):
        cid = lax.axis_index("c")
        sid = lax.axis_index("s")
        wid = cid * _NUM_SUBCORES + sid
        row0 = wid * rows_per_w

        zero16 = jnp.zeros((_LANES,), jnp.float32)

        def zero_body(k, carry):
            acc[pl.ds(k * _LANES, _LANES)] = zero16
            return carry

        lax.fori_loop(0, (2 * T) // _LANES, zero_body, 0)

        bufs = (buf0, buf1)
        sems = (sem0, sem1)
        iota = lax.iota(jnp.int32, _LANES)

        copies = [None] * nblk
        copies[0] = pltpu.async_copy(x_hbm.at[pl.ds(row0, blk), :], buf0, sem0)
        for b in range(nblk):
            if b + 1 < nblk:
                copies[b + 1] = pltpu.async_copy(
                    x_hbm.at[pl.ds(row0 + (b + 1) * blk, blk), :],
                    bufs[(b + 1) % 2],
                    sems[(b + 1) % 2],
                )
            copies[b].wait()
            buf = bufs[b % 2]

            nchunk = T // _LANES

            def row_body(r, carry):
                i = row0 + b * blk + r
                shift = T - i
                lo = jnp.maximum(0, 2 * i - T + 2)
                c0 = lo // _LANES

                @pl.when(c0 < nchunk - 1)
                def _():
                    jvec = iota + c0 * _LANES
                    xv = buf[r, pl.ds(c0 * _LANES, _LANES)]
                    plsc.addupdate_scatter(
                        acc, [jvec + shift], xv, mask=jvec >= lo
                    )

                @plsc.parallel_loop(c0 + 1, nchunk - 1, unroll=4)
                def _(k):
                    xv = buf[r, pl.ds(k * _LANES, _LANES)]
                    plsc.addupdate(acc.at[pl.ds(k * _LANES + shift, _LANES)], xv)

                jvec = iota + (nchunk - 1) * _LANES
                xv = buf[r, pl.ds((nchunk - 1) * _LANES, _LANES)]
                plsc.addupdate_scatter(
                    acc, [jvec + shift], xv,
                    mask=(jvec >= lo) & (jvec < T - 1),
                )
                return carry

            lax.fori_loop(0, blk, row_body, 0)

        pltpu.sync_copy(acc, part_hbm.at[wid])

    return sc_partials


def _combine_body(T, p_ref, o_ref):
    s = jnp.sum(p_ref[...], axis=0, keepdims=True)
    b = lax.broadcasted_iota(jnp.int32, (1, 2 * T), 1)
    cnt = jnp.where(b < T, 2 * b - T - 1, 2 * T - 1 - b)
    cnt = jnp.maximum(cnt, 0).astype(jnp.float32)
    o_ref[...] = s / cnt


def kernel(inputs):
    T = inputs.shape[0]
    partials = _make_sc_partials(T, 16)(inputs)
    out = pl.pallas_call(
        functools.partial(_combine_body, T),
        out_shape=jax.ShapeDtypeStruct((1, 2 * T), jnp.float32),
    )(partials)
    return out.reshape(2 * T)
